# baseline (speedup 1.0000x reference)
import functools

import jax
import jax.numpy as jnp
from jax import lax
from jax.experimental import pallas as pl
from jax.experimental.pallas import tpu as pltpu
from jax.experimental.pallas import tpu_sc as plsc

B = 64
N = 8192
K = 32
L = 16
NC = 2
NS = 16
ROWS_PER_W = B // (NC * NS)
GROUP = 16
NGROUPS = N // (L * GROUP)
NEG_INF = float("-inf")


def _sort16(v):
    return lax.sort(v, dimension=0)


def _merge_top32(vs, t0, t1):
    rt1 = lax.rev(t1, (0,))
    hi = jnp.maximum(vs, rt1)
    lo = jnp.minimum(vs, rt1)
    new_t1 = _sort16(hi)
    lo_s = _sort16(lo)
    rt0 = lax.rev(t0, (0,))
    hi2 = jnp.maximum(lo_s, rt0)
    new_t0 = _sort16(hi2)
    return new_t0, new_t1


def _popcnt(mask):
    return plsc.all_reduce_population_count(mask)[0]


def _kmax_body(x_hbm, out_hbm, rowa_v, rowb_v, canda_v, candb_v,
               outa_v, outb_v, ina_sem, inb_sem, outa_sem, outb_sem):
    c = lax.axis_index("c")
    s = lax.axis_index("s")
    wid = c * NS + s
    rowa = wid * ROWS_PER_W
    rowb = rowa + 1

    cpa = pltpu.async_copy(x_hbm.at[rowa], rowa_v, ina_sem)
    cpb = pltpu.async_copy(x_hbm.at[rowb], rowb_v, inb_sem)
    cpa.wait()
    cpb.wait()

    ninf_v = jnp.full((L,), NEG_INF, jnp.float32)

    def flush_one(cand_v, t0, t1, fl):
        vs = _sort16(cand_v[pl.ds(fl, L)])
        t0, t1 = _merge_top32(vs, t0, t1)
        return t0, t1, fl + L, jnp.broadcast_to(t0[0], (L,))

    def append_only(cand_v, vs, state):
        t0_, t1_, tvec_, cnt_, fl_ = state
        for v in vs:
            mj = v >= tvec_
            plsc.store_compressed(cand_v.at[pl.ds(cnt_, L)], v, mask=mj)
            cnt_ = cnt_ + _popcnt(mj)
        return t0_, t1_, tvec_, cnt_, fl_

    def flush_run(cand_v, state):
        t0_, t1_, tvec_, cnt_, fl_ = state

        def wcond(c2):
            return cnt_ - c2[2] >= L

        def wbody(c2):
            t0w, t1w, flw, _ = c2
            t0w, t1w, flw, tvw = flush_one(cand_v, t0w, t1w, flw)
            return t0w, t1w, flw, tvw

        t0_, t1_, fl_, tvec_ = lax.while_loop(
            wcond, wbody, (t0_, t1_, fl_, tvec_))
        return t0_, t1_, tvec_, cnt_, fl_

    def append_run(cand_v, vs, state):
        return flush_run(cand_v, append_only(cand_v, vs, state))

    def warm_start(row_v, cand_v):
        v0 = row_v[pl.ds(0, L)]
        v1 = row_v[pl.ds(L, L)]
        t1i = _sort16(v0)
        t0i, t1i = _merge_top32(_sort16(v1), ninf_v, t1i)
        cand_v[pl.ds(0, L)] = v0
        cand_v[pl.ds(L, L)] = v1
        tvi = jnp.broadcast_to(t0i[0], (L,))
        c32 = jnp.int32(2 * L)
        return (t0i, t1i, tvi, c32, c32)

    sta = warm_start(rowa_v, canda_v)
    stb = warm_start(rowb_v, candb_v)
    ra0 = [rowa_v[pl.ds(j * L, L)] for j in range(2, GROUP)]
    rb0 = [rowb_v[pl.ds(j * L, L)] for j in range(2, GROUP)]
    sta = append_only(canda_v, ra0, sta)
    stb = append_only(candb_v, rb0, stb)
    sta = flush_run(canda_v, sta)
    stb = flush_run(candb_v, stb)

    def scan_body(g, carry):
        sta, stb = carry
        base = g * (GROUP * L)
        va = [rowa_v[pl.ds(base + j * L, L)] for j in range(GROUP)]
        vb = [rowb_v[pl.ds(base + j * L, L)] for j in range(GROUP)]
        sta = append_only(canda_v, va, sta)
        stb = append_only(candb_v, vb, stb)
        sta = flush_run(canda_v, sta)
        stb = flush_run(candb_v, stb)
        return sta, stb

    sta, stb = lax.fori_loop(1, NGROUPS, scan_body, (sta, stb))

    def finalize(cand_v, state):
        t0, t1, tvec, cnt, fl = state
        cand_v[pl.ds(cnt, L)] = ninf_v

        def last_flush(c_):
            t0_, t1_, fl_ = c_
            t0_, t1_, fl_, _ = flush_one(cand_v, t0_, t1_, fl_)
            return t0_, t1_, fl_

        t0, t1, fl = lax.cond(cnt > fl, last_flush, lambda c_: c_,
                              (t0, t1, fl))
        return t0[0], cnt

    thra, cnta = finalize(canda_v, sta)
    thrb, cntb = finalize(candb_v, stb)

    def emit(cand_v, out_v, thr, cnt):
        q = (cnt + (L - 1)) // L

        def cnt_body(i, a):
            v = cand_v[pl.ds(i * L, L)]
            return a + _popcnt(v > thr)

        ng = lax.fori_loop(0, q, cnt_body, jnp.int32(0))
        t_eq = K - ng

        def emit_body(i, carry):
            nsel, neq = carry
            v = cand_v[pl.ds(i * L, L)]
            gt = v > thr
            eq = v == thr
            eqc = jnp.cumsum(eq.astype(jnp.int32))
            sel = jnp.logical_or(
                gt, jnp.logical_and(eq, (neq + eqc) <= t_eq))
            sel_i = sel.astype(jnp.int32)
            selc = jnp.cumsum(sel_i)
            pos = jnp.clip(nsel + selc - 1, 0, 2 * K - 1)
            plsc.store_scatter(out_v, [pos], v, mask=sel)
            return nsel + _popcnt(sel), neq + _popcnt(eq)

        lax.fori_loop(0, q, emit_body, (jnp.int32(0), jnp.int32(0)))

    emit(canda_v, outa_v, thra, cnta)
    wa = pltpu.async_copy(
        outa_v.at[pl.ds(0, K)], out_hbm.at[pl.ds(rowa * K, K)], outa_sem)
    emit(candb_v, outb_v, thrb, cntb)
    wb = pltpu.async_copy(
        outb_v.at[pl.ds(0, K)], out_hbm.at[pl.ds(rowb * K, K)], outb_sem)
    wa.wait()
    wb.wait()


@functools.lru_cache(maxsize=None)
def _build_kernel():
    mesh = plsc.VectorSubcoreMesh(
        core_axis_name="c", subcore_axis_name="s",
        num_cores=NC, num_subcores=NS)
    return pl.kernel(
        _kmax_body,
        out_type=jax.ShapeDtypeStruct((B * K,), jnp.float32),
        mesh=mesh,
        scratch_types=[
            pltpu.VMEM((N,), jnp.float32),
            pltpu.VMEM((N,), jnp.float32),
            pltpu.VMEM((N + 2 * L,), jnp.float32),
            pltpu.VMEM((N + 2 * L,), jnp.float32),
            pltpu.VMEM((2 * K,), jnp.float32),
            pltpu.VMEM((2 * K,), jnp.float32),
            pltpu.SemaphoreType.DMA,
            pltpu.SemaphoreType.DMA,
            pltpu.SemaphoreType.DMA,
            pltpu.SemaphoreType.DMA,
        ],
        compiler_params=pltpu.CompilerParams(needs_layout_passes=False),
    )


def kernel(x):
    return _build_kernel()(x).reshape(B, K)


# speedup vs baseline: 1.0519x; 1.0519x over previous
import functools

import jax
import jax.numpy as jnp
from jax import lax
from jax.experimental import pallas as pl
from jax.experimental.pallas import tpu as pltpu
from jax.experimental.pallas import tpu_sc as plsc

B = 64
N = 8192
K = 32
L = 16
NC = 2
NS = 16
ROWS_PER_W = B // (NC * NS)
GROUP = 16
NGROUPS = N // (L * GROUP)
NEG_INF = float("-inf")


def _sort16(v):
    return lax.sort(v, dimension=0)


def _merge_top32(vs, t0, t1):
    rt1 = lax.rev(t1, (0,))
    hi = jnp.maximum(vs, rt1)
    lo = jnp.minimum(vs, rt1)
    new_t1 = _sort16(hi)
    lo_s = _sort16(lo)
    rt0 = lax.rev(t0, (0,))
    hi2 = jnp.maximum(lo_s, rt0)
    new_t0 = _sort16(hi2)
    return new_t0, new_t1


def _popcnt(mask):
    return plsc.all_reduce_population_count(mask)[0]


def _kmax_body(x_hbm, out_hbm, rowa_v, rowb_v, canda_v, candb_v,
               outa_v, outb_v, ina_sem, inb_sem, outa_sem, outb_sem):
    c = lax.axis_index("c")
    s = lax.axis_index("s")
    wid = c * NS + s
    rowa = wid * ROWS_PER_W
    rowb = rowa + 1

    cpa = pltpu.async_copy(x_hbm.at[rowa], rowa_v, ina_sem)
    cpb = pltpu.async_copy(x_hbm.at[rowb], rowb_v, inb_sem)
    cpa.wait()
    cpb.wait()

    ninf_v = jnp.full((L,), NEG_INF, jnp.float32)

    def flush_one(cand_v, t0, t1, fl):
        vs = _sort16(cand_v[pl.ds(fl, L)])
        t0, t1 = _merge_top32(vs, t0, t1)
        return t0, t1, fl + L, jnp.broadcast_to(t0[0], (L,))

    def append_only(cand_v, vs, state):
        t0_, t1_, tvec_, cnt_, fl_ = state
        for v in vs:
            mj = v >= tvec_
            plsc.store_compressed(cand_v.at[pl.ds(cnt_, L)], v, mask=mj)
            cnt_ = cnt_ + _popcnt(mj)
        return t0_, t1_, tvec_, cnt_, fl_

    def flush_run(cand_v, state):
        t0_, t1_, tvec_, cnt_, fl_ = state

        def wcond(c2):
            return cnt_ - c2[2] >= L

        def wbody(c2):
            t0w, t1w, flw, _ = c2
            t0w, t1w, flw, tvw = flush_one(cand_v, t0w, t1w, flw)
            return t0w, t1w, flw, tvw

        t0_, t1_, fl_, tvec_ = lax.while_loop(
            wcond, wbody, (t0_, t1_, fl_, tvec_))
        return t0_, t1_, tvec_, cnt_, fl_

    def append_run(cand_v, vs, state):
        return flush_run(cand_v, append_only(cand_v, vs, state))

    z = jnp.int32(0)
    sta = (ninf_v, ninf_v, ninf_v, z, z)
    stb = (ninf_v, ninf_v, ninf_v, z, z)

    def scan_body(g, carry):
        sta, stb = carry
        base = g * (GROUP * L)
        va = [rowa_v[pl.ds(base + j * L, L)] for j in range(GROUP)]
        vb = [rowb_v[pl.ds(base + j * L, L)] for j in range(GROUP)]
        sta = append_only(canda_v, va, sta)
        stb = append_only(candb_v, vb, stb)
        sta = flush_run(canda_v, sta)
        stb = flush_run(candb_v, stb)
        return sta, stb

    sta, stb = lax.fori_loop(0, NGROUPS, scan_body, (sta, stb))

    def finalize(cand_v, state):
        t0, t1, tvec, cnt, fl = state
        cand_v[pl.ds(cnt, L)] = ninf_v

        def last_flush(c_):
            t0_, t1_, fl_ = c_
            t0_, t1_, fl_, _ = flush_one(cand_v, t0_, t1_, fl_)
            return t0_, t1_, fl_

        t0, t1, fl = lax.cond(cnt > fl, last_flush, lambda c_: c_,
                              (t0, t1, fl))
        thr = t0[0]
        ng = _popcnt(t0 > thr) + _popcnt(t1 > thr)
        return thr, cnt, ng

    thra, cnta, nga = finalize(canda_v, sta)
    thrb, cntb, ngb = finalize(candb_v, stb)

    def emit(cand_v, out_v, thr, cnt, ng):
        q = (cnt + (L - 1)) // L
        t_eq = K - ng

        def emit_body(i, carry):
            nsel, neq = carry
            v = cand_v[pl.ds(i * L, L)]
            gt = v > thr
            eq = v == thr
            eqc = jnp.cumsum(eq.astype(jnp.int32))
            sel = jnp.logical_or(
                gt, jnp.logical_and(eq, (neq + eqc) <= t_eq))
            sel_i = sel.astype(jnp.int32)
            selc = jnp.cumsum(sel_i)
            pos = jnp.clip(nsel + selc - 1, 0, 2 * K - 1)
            plsc.store_scatter(out_v, [pos], v, mask=sel)
            return nsel + _popcnt(sel), neq + _popcnt(eq)

        lax.fori_loop(0, q, emit_body, (jnp.int32(0), jnp.int32(0)))

    emit(canda_v, outa_v, thra, cnta, nga)
    wa = pltpu.async_copy(
        outa_v.at[pl.ds(0, K)], out_hbm.at[pl.ds(rowa * K, K)], outa_sem)
    emit(candb_v, outb_v, thrb, cntb, ngb)
    wb = pltpu.async_copy(
        outb_v.at[pl.ds(0, K)], out_hbm.at[pl.ds(rowb * K, K)], outb_sem)
    wa.wait()
    wb.wait()


@functools.lru_cache(maxsize=None)
def _build_kernel():
    mesh = plsc.VectorSubcoreMesh(
        core_axis_name="c", subcore_axis_name="s",
        num_cores=NC, num_subcores=NS)
    return pl.kernel(
        _kmax_body,
        out_type=jax.ShapeDtypeStruct((B * K,), jnp.float32),
        mesh=mesh,
        scratch_types=[
            pltpu.VMEM((N,), jnp.float32),
            pltpu.VMEM((N,), jnp.float32),
            pltpu.VMEM((N + 2 * L,), jnp.float32),
            pltpu.VMEM((N + 2 * L,), jnp.float32),
            pltpu.VMEM((2 * K,), jnp.float32),
            pltpu.VMEM((2 * K,), jnp.float32),
            pltpu.SemaphoreType.DMA,
            pltpu.SemaphoreType.DMA,
            pltpu.SemaphoreType.DMA,
            pltpu.SemaphoreType.DMA,
        ],
        compiler_params=pltpu.CompilerParams(needs_layout_passes=False),
    )


def kernel(x):
    return _build_kernel()(x).reshape(B, K)
